# Initial kernel scaffold; baseline (speedup 1.0000x reference)
import jax
import jax.numpy as jnp
from jax.experimental import pallas as pl


def kernel(q_rel, layer_input, edges, nodes, n_ent, rela_embed, Ws_attn, Wr_attn, Wqr_attn_W, Wqr_attn_b, w_alpha_W, w_alpha_b, W_h):
    raise NotImplementedError("write your pallas kernel here")


# speedup vs baseline: 1.4668x; 1.4668x over previous
import functools

import jax
import jax.numpy as jnp
from jax import lax
from jax.experimental import pallas as pl
from jax.experimental.pallas import tpu as pltpu
from jax.experimental.pallas import tpu_sc as plsc

N_NODES = 10000
N_EDGES = 160000
D = 256
MP = 10240
NC, NS = 2, 16
NW = NC * NS

EPW = N_EDGES // NW
B1 = 40
NCH1 = EPW // B1

EPT = N_EDGES // NS
B2 = 80
NCH2 = EPT // B2
RPT = N_NODES // NS
RC = 125
NRC = RPT // RC


def _mm_kernel(x_ref, w_ref, o_ref):
    o_ref[...] = jnp.dot(x_ref[...], w_ref[...],
                         preferred_element_type=jnp.float32)


def _mm_bias_kernel(x_ref, w_ref, b_ref, o_ref):
    o_ref[...] = jnp.dot(x_ref[...], w_ref[...],
                         preferred_element_type=jnp.float32) + b_ref[0:1, :]


def _mm(x, w):
    m, k = x.shape
    n = w.shape[1]
    bm = 1024
    return pl.pallas_call(
        _mm_kernel,
        grid=(m // bm,),
        in_specs=[pl.BlockSpec((bm, k), lambda i: (i, 0)),
                  pl.BlockSpec((k, n), lambda i: (0, 0))],
        out_specs=pl.BlockSpec((bm, n), lambda i: (i, 0)),
        out_shape=jax.ShapeDtypeStruct((m, n), jnp.float32),
    )(x, w)


def _mm_bias(x, w, b):
    m, k = x.shape
    n = w.shape[1]
    bm = 1024
    b8 = jnp.zeros((8, n), jnp.float32).at[0].set(b)
    return pl.pallas_call(
        _mm_bias_kernel,
        grid=(m // bm,),
        in_specs=[pl.BlockSpec((bm, k), lambda i: (i, 0)),
                  pl.BlockSpec((k, n), lambda i: (0, 0)),
                  pl.BlockSpec((8, n), lambda i: (0, 0))],
        out_specs=pl.BlockSpec((bm, n), lambda i: (i, 0)),
        out_shape=jax.ShapeDtypeStruct((m, n), jnp.float32),
    )(x, w, b8)


def _mm_colsplit(x, w):
    m, k = x.shape
    bm = 1024
    nb = m // bm
    return pl.pallas_call(
        _mm_kernel,
        grid=(nb, 2),
        in_specs=[pl.BlockSpec((bm, k), lambda i, j: (i, 0)),
                  pl.BlockSpec((k, 128), lambda i, j: (0, j))],
        out_specs=pl.BlockSpec((bm, 128), lambda i, j: (j * nb + i, 0)),
        out_shape=jax.ShapeDtypeStruct((2 * m, 128), jnp.float32),
    )(x, w)


def _alpha_body(ps, pr, pq, sub, rel, ridx, qrel, wcat, alpha_out,
                isub, irel, irx, iqr, ra, rb, rc_, wv, dots, sem):
    c = lax.axis_index("c")
    s = lax.axis_index("s")
    wid = s * NC + c
    base0 = wid * EPW
    pltpu.sync_copy(wcat, wv)
    b2 = wv[pl.ds(256, 16)][0]
    wregs = [wv[pl.ds(j * 16, 16)] for j in range(16)]
    lane = lax.iota(jnp.int32, 16)
    m15 = lane == 15

    def chunk_body(kk, carry):
        base = base0 + kk * B1
        pltpu.sync_copy(sub.at[pl.ds(base, B1)], isub)
        pltpu.sync_copy(rel.at[pl.ds(base, B1)], irel)
        pltpu.sync_copy(ridx.at[pl.ds(base, B1)], irx)
        pltpu.async_copy(qrel.at[irx], iqr, sem).wait()
        pltpu.async_copy(ps.at[isub], ra, sem).wait()
        pltpu.async_copy(pr.at[irel], rb, sem).wait()
        pltpu.async_copy(pq.at[iqr], rc_, sem).wait()

        def edge_body(e, cy):
            acc = jnp.zeros((16,), jnp.float32)
            for j in range(16):
                sl = pl.ds(j * 16, 16)
                v = jnp.maximum(ra[e, sl] + rb[e, sl] + rc_[e, sl], 0.0)
                acc = acc + v * wregs[j]
            tot = plsc.cumsum(acc)
            plsc.store_scatter(dots, [jnp.full((16,), e, jnp.int32)],
                               tot, mask=m15)
            return cy

        lax.fori_loop(0, B1, edge_body, 0)
        for g in range(3):
            sl = pl.ds(g * 16, 16)
            v = dots[sl]
            dots[sl] = 1.0 / (1.0 + jnp.exp(-(v + b2)))
        pltpu.sync_copy(dots.at[pl.ds(0, B1)], alpha_out.at[pl.ds(base, B1)])
        return carry

    lax.fori_loop(0, NCH1, chunk_body, 0)


def _alpha_pass(ps, pr, pq, sub, rel, ridx, qrel, wcat):
    mesh = plsc.VectorSubcoreMesh(core_axis_name="c", subcore_axis_name="s")
    f = pl.kernel(
        _alpha_body,
        out_type=jax.ShapeDtypeStruct((N_EDGES,), jnp.float32),
        mesh=mesh,
        scratch_types=[
            pltpu.VMEM((B1,), jnp.int32),
            pltpu.VMEM((B1,), jnp.int32),
            pltpu.VMEM((B1,), jnp.int32),
            pltpu.VMEM((B1,), jnp.int32),
            pltpu.VMEM((B1, D), jnp.float32),
            pltpu.VMEM((B1, D), jnp.float32),
            pltpu.VMEM((B1, D), jnp.float32),
            pltpu.VMEM((272,), jnp.float32),
            pltpu.VMEM((48,), jnp.float32),
            pltpu.SemaphoreType.DMA,
        ],
        compiler_params=pltpu.CompilerParams(needs_layout_passes=False, use_tc_tiling_on_sc=False),
    )
    return f(ps, pr, pq, sub, rel, ridx, qrel, wcat)


def _agg_body(ls, lr, sub, rel, obj, alpha, out,
              shared, isub, irel, iobj, abuf, ga, gb, obuf, sem):
    c = lax.axis_index("c")
    s = lax.axis_index("s")
    coff = c * 128
    roff = c * MP

    def zrow(i, cy):
        for j in range(8):
            obuf[i, pl.ds(j * 16, 16)] = jnp.zeros((16,), jnp.float32)
        return cy
    lax.fori_loop(0, RC, zrow, 0)
    for t in range(NRC):
        pltpu.sync_copy(obuf, shared.at[pl.ds(s * RPT + t * RC, RC)])
    plsc.subcore_barrier()

    def chunk_body(kk, carry):
        base = s * EPT + kk * B2
        pltpu.sync_copy(sub.at[pl.ds(base, B2)], isub)
        pltpu.sync_copy(rel.at[pl.ds(base, B2)], irel)
        pltpu.sync_copy(obj.at[pl.ds(base, B2)], iobj)
        pltpu.sync_copy(alpha.at[pl.ds(base, B2)], abuf)
        for i in range(B2 // 16):
            sl = pl.ds(i * 16, 16)
            isub[sl] = isub[sl] + roff
            irel[sl] = irel[sl] + roff
        pltpu.async_copy(ls.at[isub], ga, sem).wait()
        pltpu.async_copy(lr.at[irel], gb, sem).wait()

        def group_body(g, cy):
            av = abuf[pl.ds(g * 16, 16)]
            for l in range(16):
                e = g * 16 + l
                a = av[l]
                for j in range(8):
                    sl = pl.ds(j * 16, 16)
                    ga[e, sl] = (ga[e, sl] + gb[e, sl]) * a
            return cy

        lax.fori_loop(0, B2 // 16, group_body, 0)
        pltpu.sync_copy(ga, shared.at[iobj], add=True)
        return carry

    lax.fori_loop(0, NCH2, chunk_body, 0)
    plsc.subcore_barrier()

    for t in range(NRC):
        r0 = s * RPT + t * RC
        pltpu.sync_copy(shared.at[pl.ds(r0, RC)], obuf)

        def rrow(i, cy):
            for j in range(8):
                sl = pl.ds(j * 16, 16)
                obuf[i, sl] = jnp.maximum(obuf[i, sl], 0.0)
            return cy
        lax.fori_loop(0, RC, rrow, 0)
        pltpu.sync_copy(obuf, out.at[pl.ds(r0, RC), pl.ds(coff, 128)])


def _agg_pass(ls, lr, sub, rel, obj, alpha):
    mesh = plsc.VectorSubcoreMesh(core_axis_name="c", subcore_axis_name="s")
    f = pl.kernel(
        _agg_body,
        out_type=jax.ShapeDtypeStruct((N_NODES, D), jnp.float32),
        mesh=mesh,
        scratch_types=[
            pltpu.VMEM_SHARED((N_NODES, 128), jnp.float32),
            pltpu.VMEM((B2,), jnp.int32),
            pltpu.VMEM((B2,), jnp.int32),
            pltpu.VMEM((B2,), jnp.int32),
            pltpu.VMEM((B2,), jnp.float32),
            pltpu.VMEM((B2, 128), jnp.float32),
            pltpu.VMEM((B2, 128), jnp.float32),
            pltpu.VMEM((RC, 128), jnp.float32),
            pltpu.SemaphoreType.DMA,
        ],
        compiler_params=pltpu.CompilerParams(needs_layout_passes=False, use_tc_tiling_on_sc=False),
    )
    return f(ls, lr, sub, rel, obj, alpha)


def kernel(q_rel, layer_input, edges, nodes, n_ent, rela_embed,
           Ws_attn, Wr_attn, Wqr_attn_W, Wqr_attn_b,
           w_alpha_W, w_alpha_b, W_h):
    sub = edges[:, 4]
    rel = edges[:, 2]
    obj = edges[:, 5]
    ridx = edges[:, 0]

    xl = jnp.pad(layer_input, ((0, MP - layer_input.shape[0]), (0, 0)))
    xr = jnp.pad(rela_embed, ((0, MP - rela_embed.shape[0]), (0, 0)))

    ps = _mm(xl, Ws_attn)
    pr = _mm(xr, Wr_attn)
    pq = _mm_bias(xr, Wqr_attn_W, Wqr_attn_b)
    ls = _mm_colsplit(xl, W_h)
    lr = _mm_colsplit(xr, W_h)

    wcat = jnp.concatenate(
        [w_alpha_W[:, 0], w_alpha_b, jnp.zeros((15,), jnp.float32)])

    alpha = _alpha_pass(ps, pr, pq, sub, rel, ridx, q_rel, wcat)
    return _agg_pass(ls, lr, sub, rel, obj, alpha)


# speedup vs baseline: 3.5553x; 2.4239x over previous
import functools

import jax
import jax.numpy as jnp
from jax import lax
from jax.experimental import pallas as pl
from jax.experimental.pallas import tpu as pltpu
from jax.experimental.pallas import tpu_sc as plsc

N_NODES = 10000
N_EDGES = 160000
D = 256
MP = 10240
NC, NS = 2, 16
NW = NC * NS

EPW = N_EDGES // NW
B1 = 40
NCH1 = EPW // B1

EPT = N_EDGES // NS
B2 = 40
WIN = 2000
NWIN = EPT // WIN
CPW = WIN // B2
RPT = N_NODES // NS
RC = 25
NRC = RPT // RC


def _mm_kernel(x_ref, w_ref, o_ref):
    o_ref[...] = jnp.dot(x_ref[...], w_ref[...],
                         preferred_element_type=jnp.float32)


def _mm_bias_kernel(x_ref, w_ref, b_ref, o_ref):
    o_ref[...] = jnp.dot(x_ref[...], w_ref[...],
                         preferred_element_type=jnp.float32) + b_ref[0:1, :]


def _mm(x, w):
    m, k = x.shape
    n = w.shape[1]
    bm = 1024
    return pl.pallas_call(
        _mm_kernel,
        grid=(m // bm,),
        in_specs=[pl.BlockSpec((bm, k), lambda i: (i, 0)),
                  pl.BlockSpec((k, n), lambda i: (0, 0))],
        out_specs=pl.BlockSpec((bm, n), lambda i: (i, 0)),
        out_shape=jax.ShapeDtypeStruct((m, n), jnp.float32),
    )(x, w)


def _mm_bias(x, w, b):
    m, k = x.shape
    n = w.shape[1]
    bm = 1024
    b8 = jnp.zeros((8, n), jnp.float32).at[0].set(b)
    return pl.pallas_call(
        _mm_bias_kernel,
        grid=(m // bm,),
        in_specs=[pl.BlockSpec((bm, k), lambda i: (i, 0)),
                  pl.BlockSpec((k, n), lambda i: (0, 0)),
                  pl.BlockSpec((8, n), lambda i: (0, 0))],
        out_specs=pl.BlockSpec((bm, n), lambda i: (i, 0)),
        out_shape=jax.ShapeDtypeStruct((m, n), jnp.float32),
    )(x, w, b8)


def _mm_colsplit(x, w):
    m, k = x.shape
    bm = 1024
    nb = m // bm
    return pl.pallas_call(
        _mm_kernel,
        grid=(nb, 2),
        in_specs=[pl.BlockSpec((bm, k), lambda i, j: (i, 0)),
                  pl.BlockSpec((k, 128), lambda i, j: (0, j))],
        out_specs=pl.BlockSpec((bm, 128), lambda i, j: (j * nb + i, 0)),
        out_shape=jax.ShapeDtypeStruct((2 * m, 128), jnp.float32),
    )(x, w)


def _alpha_body(ps, pr, pq, sub, rel, ridx, qrel, wcat, alpha_out,
                isub, irel, irx, iqr, ra0, rb0, rc0, ra1, rb1, rc1,
                wv, dots, semq, sa, sb):
    c = lax.axis_index("c")
    s = lax.axis_index("s")
    wid = s * NC + c
    base0 = wid * EPW
    pltpu.sync_copy(wcat, wv)
    b2 = wv[pl.ds(256, 16)][0]
    wregs = [wv[pl.ds(j * 16, 16)] for j in range(16)]
    lane = lax.iota(jnp.int32, 16)
    m15 = lane == 15

    pltpu.sync_copy(sub.at[pl.ds(base0, EPW)], isub)
    pltpu.sync_copy(rel.at[pl.ds(base0, EPW)], irel)
    pltpu.sync_copy(ridx.at[pl.ds(base0, EPW)], irx)
    def qgrp(g, cy):
        for i in range(5):
            st = (g * 5 + i) * B1
            pltpu.async_copy(qrel.at[irx.at[pl.ds(st, B1)]],
                             iqr.at[pl.ds(st, B1)], semq)
        for i in range(5):
            st = (g * 5 + i) * B1
            pltpu.make_async_copy(qrel.at[irx.at[pl.ds(st, B1)]],
                                  iqr.at[pl.ds(st, B1)], semq).wait()
        return cy
    lax.fori_loop(0, NCH1 // 5, qgrp, 0)

    def fire(kk, bufs, sem):
        st = kk * B1
        ra, rb, rc_ = bufs
        pltpu.async_copy(ps.at[isub.at[pl.ds(st, B1)]], ra, sem)
        pltpu.async_copy(pr.at[irel.at[pl.ds(st, B1)]], rb, sem)
        pltpu.async_copy(pq.at[iqr.at[pl.ds(st, B1)]], rc_, sem)

    def drain(kk, bufs, sem):
        st = kk * B1
        ra, rb, rc_ = bufs
        pltpu.make_async_copy(ps.at[isub.at[pl.ds(st, B1)]], ra, sem).wait()
        pltpu.make_async_copy(pr.at[irel.at[pl.ds(st, B1)]], rb, sem).wait()
        pltpu.make_async_copy(pq.at[iqr.at[pl.ds(st, B1)]], rc_, sem).wait()

    def compute(kk, bufs):
        ra, rb, rc_ = bufs

        def edge_body(e, cy):
            acc = jnp.zeros((16,), jnp.float32)
            for j in range(16):
                sl = pl.ds(j * 16, 16)
                v = jnp.maximum(ra[e, sl] + rb[e, sl] + rc_[e, sl], 0.0)
                acc = acc + v * wregs[j]
            tot = plsc.cumsum(acc)
            plsc.store_scatter(dots, [jnp.full((16,), kk * B1 + e, jnp.int32)],
                               tot, mask=m15)
            return cy

        lax.fori_loop(0, B1, edge_body, 0)

    set0 = (ra0, rb0, rc0)
    set1 = (ra1, rb1, rc1)
    fire(0, set0, sa)

    def pipe(g, cy):
        k0 = g * 2
        drain(k0, set0, sa)
        fire(k0 + 1, set1, sb)
        compute(k0, set0)
        drain(k0 + 1, set1, sb)

        @pl.when(g < NCH1 // 2 - 1)
        def _():
            fire(k0 + 2, set0, sa)
        compute(k0 + 1, set1)
        return cy

    lax.fori_loop(0, NCH1 // 2, pipe, 0)
    fire(NCH1 - 1, set0, sa)
    drain(NCH1 - 1, set0, sa)
    compute(NCH1 - 1, set0)

    def sig(g, cy):
        sl = pl.ds(g * 16, 16)
        v = dots[sl]
        dots[sl] = 1.0 / (1.0 + jnp.exp(-(v + b2)))
        return cy
    lax.fori_loop(0, (EPW + 8) // 16, sig, 0)
    pltpu.sync_copy(dots.at[pl.ds(0, EPW)], alpha_out.at[pl.ds(base0, EPW)])


def _alpha_pass(ps, pr, pq, sub, rel, ridx, qrel, wcat):
    mesh = plsc.VectorSubcoreMesh(core_axis_name="c", subcore_axis_name="s")
    f = pl.kernel(
        _alpha_body,
        out_type=jax.ShapeDtypeStruct((N_EDGES,), jnp.float32),
        mesh=mesh,
        scratch_types=[
            pltpu.VMEM((EPW,), jnp.int32),
            pltpu.VMEM((EPW,), jnp.int32),
            pltpu.VMEM((EPW,), jnp.int32),
            pltpu.VMEM((EPW,), jnp.int32),
            pltpu.VMEM((B1, D), jnp.float32),
            pltpu.VMEM((B1, D), jnp.float32),
            pltpu.VMEM((B1, D), jnp.float32),
            pltpu.VMEM((B1, D), jnp.float32),
            pltpu.VMEM((B1, D), jnp.float32),
            pltpu.VMEM((B1, D), jnp.float32),
            pltpu.VMEM((272,), jnp.float32),
            pltpu.VMEM((EPW + 8,), jnp.float32),
            pltpu.SemaphoreType.DMA,
            pltpu.SemaphoreType.DMA,
            pltpu.SemaphoreType.DMA,
        ],
        compiler_params=pltpu.CompilerParams(needs_layout_passes=False, use_tc_tiling_on_sc=False),
    )
    return f(ps, pr, pq, sub, rel, ridx, qrel, wcat)


def _agg_body(ls, lr, sub, rel, obj, alpha, out,
              shared, isub, irel, iobj, abuf, ga, gb, gc, gd, obuf, sa, sb):
    c = lax.axis_index("c")
    s = lax.axis_index("s")
    coff = c * 128
    roff = c * MP

    def zrow(i, cy):
        for j in range(8):
            obuf[i, pl.ds(j * 16, 16)] = jnp.zeros((16,), jnp.float32)
        return cy
    lax.fori_loop(0, RC, zrow, 0)

    def zcopy(t, cy):
        pltpu.sync_copy(obuf, shared.at[pl.ds(s * RPT + t * RC, RC)])
        return cy
    lax.fori_loop(0, NRC, zcopy, 0)
    plsc.subcore_barrier()

    base0 = s * EPT

    def load_window(w):
        wb = base0 + w * WIN
        pltpu.sync_copy(sub.at[pl.ds(wb, WIN)], isub)
        pltpu.sync_copy(rel.at[pl.ds(wb, WIN)], irel)
        pltpu.sync_copy(obj.at[pl.ds(wb, WIN)], iobj)
        pltpu.sync_copy(alpha.at[pl.ds(wb, WIN)], abuf.at[pl.ds(0, WIN)])

        def offs(i, cy):
            sl = pl.ds(i * 16, 16)
            isub[sl] = isub[sl] + roff
            irel[sl] = irel[sl] + roff
            return cy
        lax.fori_loop(0, WIN // 16, offs, 0)

    def fire(kk, bufs, sem):
        st = kk * B2
        ga_, gb_ = bufs
        pltpu.async_copy(ls.at[isub.at[pl.ds(st, B2)]], ga_, sem)
        pltpu.async_copy(lr.at[irel.at[pl.ds(st, B2)]], gb_, sem)

    def drain(kk, bufs, sem):
        st = kk * B2
        ga_, gb_ = bufs
        pltpu.make_async_copy(ls.at[isub.at[pl.ds(st, B2)]], ga_, sem).wait()
        pltpu.make_async_copy(lr.at[irel.at[pl.ds(st, B2)]], gb_, sem).wait()

    def work(kk, bufs):
        st = kk * B2
        ga_, gb_ = bufs

        def group_body(g, cy):
            av = abuf[pl.ds(st + g * 16, 16)]
            for l in range(16):
                e = g * 16 + l
                a = av[l]
                for j in range(8):
                    sl = pl.ds(j * 16, 16)
                    ga_[e, sl] = (ga_[e, sl] + gb_[e, sl]) * a
            return cy

        lax.fori_loop(0, B2 // 16, group_body, 0)
        av = abuf[pl.ds(st + 32, 16)]
        for l in range(8):
            e = 32 + l
            a = av[l]
            for j in range(8):
                sl = pl.ds(j * 16, 16)
                ga_[e, sl] = (ga_[e, sl] + gb_[e, sl]) * a
        pltpu.sync_copy(ga_, shared.at[iobj.at[pl.ds(st, B2)]], add=True)

    set0 = (ga, gb)
    set1 = (gc, gd)

    def window_body(w, wcy):
        load_window(w)
        fire(0, set0, sa)

        def pipe(k, cy):
            even = lax.rem(k, 2) == 0

            @pl.when(even)
            def _():
                drain(k, set0, sa)

                @pl.when(k < CPW - 1)
                def _():
                    fire(k + 1, set1, sb)
                work(k, set0)

            @pl.when(jnp.logical_not(even))
            def _():
                drain(k, set1, sb)

                @pl.when(k < CPW - 1)
                def _():
                    fire(k + 1, set0, sa)
                work(k, set1)
            return cy

        lax.fori_loop(0, CPW, pipe, 0)
        return wcy

    lax.fori_loop(0, NWIN, window_body, 0)
    plsc.subcore_barrier()

    def ocopy(t, cy):
        r0 = s * RPT + t * RC
        pltpu.sync_copy(shared.at[pl.ds(r0, RC)], obuf)

        def rrow(i, icy):
            for j in range(8):
                sl = pl.ds(j * 16, 16)
                obuf[i, sl] = jnp.maximum(obuf[i, sl], 0.0)
            return icy
        lax.fori_loop(0, RC, rrow, 0)
        pltpu.sync_copy(obuf, out.at[pl.ds(r0, RC), pl.ds(coff, 128)])
        return cy
    lax.fori_loop(0, NRC, ocopy, 0)


def _agg_pass(ls, lr, sub, rel, obj, alpha):
    mesh = plsc.VectorSubcoreMesh(core_axis_name="c", subcore_axis_name="s")
    f = pl.kernel(
        _agg_body,
        out_type=jax.ShapeDtypeStruct((N_NODES, D), jnp.float32),
        mesh=mesh,
        scratch_types=[
            pltpu.VMEM_SHARED((N_NODES, 128), jnp.float32),
            pltpu.VMEM((WIN,), jnp.int32),
            pltpu.VMEM((WIN,), jnp.int32),
            pltpu.VMEM((WIN,), jnp.int32),
            pltpu.VMEM((WIN + 16,), jnp.float32),
            pltpu.VMEM((B2, 128), jnp.float32),
            pltpu.VMEM((B2, 128), jnp.float32),
            pltpu.VMEM((B2, 128), jnp.float32),
            pltpu.VMEM((B2, 128), jnp.float32),
            pltpu.VMEM((RC, 128), jnp.float32),
            pltpu.SemaphoreType.DMA,
            pltpu.SemaphoreType.DMA,
        ],
        compiler_params=pltpu.CompilerParams(needs_layout_passes=False, use_tc_tiling_on_sc=False),
    )
    return f(ls, lr, sub, rel, obj, alpha)


def kernel(q_rel, layer_input, edges, nodes, n_ent, rela_embed,
           Ws_attn, Wr_attn, Wqr_attn_W, Wqr_attn_b,
           w_alpha_W, w_alpha_b, W_h):
    sub = edges[:, 4]
    rel = edges[:, 2]
    obj = edges[:, 5]
    ridx = edges[:, 0]

    xl = jnp.pad(layer_input, ((0, MP - layer_input.shape[0]), (0, 0)))
    xr = jnp.pad(rela_embed, ((0, MP - rela_embed.shape[0]), (0, 0)))

    ps = _mm(xl, Ws_attn)
    pr = _mm(xr, Wr_attn)
    pq = _mm_bias(xr, Wqr_attn_W, Wqr_attn_b)
    ls = _mm_colsplit(xl, W_h)
    lr = _mm_colsplit(xr, W_h)

    wcat = jnp.concatenate(
        [w_alpha_W[:, 0], w_alpha_b, jnp.zeros((15,), jnp.float32)])

    alpha = _alpha_pass(ps, pr, pq, sub, rel, ridx, q_rel, wcat)
    return _agg_pass(ls, lr, sub, rel, obj, alpha)


# speedup vs baseline: 3.6732x; 1.0332x over previous
import functools

import jax
import jax.numpy as jnp
from jax import lax
from jax.experimental import pallas as pl
from jax.experimental.pallas import tpu as pltpu
from jax.experimental.pallas import tpu_sc as plsc

N_NODES = 10000
N_QUERIES = 10000
N_EDGES = 160000
D = 256
MP = 10240
NC, NS = 2, 16
NW = NC * NS

EPW = N_EDGES // NW
B1 = 40
NCH1 = EPW // B1

EPT = N_EDGES // NS
B2 = 40
WIN = 2000
NWIN = EPT // WIN
CPW = WIN // B2
RPT = N_NODES // NS
RC = 25
NRC = RPT // RC


def _mm_kernel(x_ref, w_ref, o_ref):
    o_ref[...] = jnp.dot(x_ref[...], w_ref[...],
                         preferred_element_type=jnp.float32)


def _mm_bias_kernel(x_ref, w_ref, b_ref, o_ref):
    o_ref[...] = jnp.dot(x_ref[...], w_ref[...],
                         preferred_element_type=jnp.float32) + b_ref[0:1, :]


def _mm(x, w):
    m, k = x.shape
    n = w.shape[1]
    bm = 1024
    return pl.pallas_call(
        _mm_kernel,
        grid=(m // bm,),
        in_specs=[pl.BlockSpec((bm, k), lambda i: (i, 0)),
                  pl.BlockSpec((k, n), lambda i: (0, 0))],
        out_specs=pl.BlockSpec((bm, n), lambda i: (i, 0)),
        out_shape=jax.ShapeDtypeStruct((m, n), jnp.float32),
    )(x, w)


def _mm_bias(x, w, b):
    m, k = x.shape
    n = w.shape[1]
    bm = 1024
    b8 = jnp.zeros((8, n), jnp.float32).at[0].set(b)
    return pl.pallas_call(
        _mm_bias_kernel,
        grid=(m // bm,),
        in_specs=[pl.BlockSpec((bm, k), lambda i: (i, 0)),
                  pl.BlockSpec((k, n), lambda i: (0, 0)),
                  pl.BlockSpec((8, n), lambda i: (0, 0))],
        out_specs=pl.BlockSpec((bm, n), lambda i: (i, 0)),
        out_shape=jax.ShapeDtypeStruct((m, n), jnp.float32),
    )(x, w, b8)


def _mm_colsplit(x, w):
    m, k = x.shape
    bm = 1024
    nb = m // bm
    return pl.pallas_call(
        _mm_kernel,
        grid=(nb, 2),
        in_specs=[pl.BlockSpec((bm, k), lambda i, j: (i, 0)),
                  pl.BlockSpec((k, 128), lambda i, j: (0, j))],
        out_specs=pl.BlockSpec((bm, 128), lambda i, j: (j * nb + i, 0)),
        out_shape=jax.ShapeDtypeStruct((2 * m, 128), jnp.float32),
    )(x, w)


def _alpha_body(ps, pr, pq, sub, rel, ridx, qrel, wcat, alpha_out,
                isub, irel, irx, iqr, qtab, ra0, rb0, rc0, ra1, rb1, rc1,
                wv, dots, sa, sb):
    c = lax.axis_index("c")
    s = lax.axis_index("s")
    wid = s * NC + c
    base0 = wid * EPW
    pltpu.sync_copy(wcat, wv)
    b2 = wv[pl.ds(256, 16)][0]
    wregs = [wv[pl.ds(j * 16, 16)] for j in range(16)]
    lane = lax.iota(jnp.int32, 16)
    m15 = lane == 15

    irx[pl.ds(EPW - 8, 16)] = jnp.zeros((16,), jnp.int32)
    pltpu.sync_copy(sub.at[pl.ds(base0, EPW)], isub)
    pltpu.sync_copy(rel.at[pl.ds(base0, EPW)], irel)
    pltpu.sync_copy(ridx.at[pl.ds(base0, EPW)], irx.at[pl.ds(0, EPW)])
    pltpu.sync_copy(qrel, qtab)

    def qcomp(g, cy):
        sl = pl.ds(g * 16, 16)
        iqr[sl] = plsc.load_gather(qtab, [irx[sl]])
        return cy
    lax.fori_loop(0, (EPW + 8) // 16, qcomp, 0)

    def fire(kk, bufs, sem):
        st = kk * B1
        ra, rb, rc_ = bufs
        pltpu.async_copy(ps.at[isub.at[pl.ds(st, B1)]], ra, sem)
        pltpu.async_copy(pr.at[irel.at[pl.ds(st, B1)]], rb, sem)
        pltpu.async_copy(pq.at[iqr.at[pl.ds(st, B1)]], rc_, sem)

    def drain(kk, bufs, sem):
        st = kk * B1
        ra, rb, rc_ = bufs
        pltpu.make_async_copy(ps.at[isub.at[pl.ds(st, B1)]], ra, sem).wait()
        pltpu.make_async_copy(pr.at[irel.at[pl.ds(st, B1)]], rb, sem).wait()
        pltpu.make_async_copy(pq.at[iqr.at[pl.ds(st, B1)]], rc_, sem).wait()

    def compute(kk, bufs):
        ra, rb, rc_ = bufs

        def edge_body(e, cy):
            acc = jnp.zeros((16,), jnp.float32)
            for j in range(16):
                sl = pl.ds(j * 16, 16)
                v = jnp.maximum(ra[e, sl] + rb[e, sl] + rc_[e, sl], 0.0)
                acc = acc + v * wregs[j]
            tot = plsc.cumsum(acc)
            plsc.store_scatter(dots, [jnp.full((16,), kk * B1 + e, jnp.int32)],
                               tot, mask=m15)
            return cy

        lax.fori_loop(0, B1, edge_body, 0)

    set0 = (ra0, rb0, rc0)
    set1 = (ra1, rb1, rc1)
    fire(0, set0, sa)

    def pipe(g, cy):
        k0 = g * 2
        drain(k0, set0, sa)
        fire(k0 + 1, set1, sb)
        compute(k0, set0)
        drain(k0 + 1, set1, sb)

        @pl.when(g < NCH1 // 2 - 1)
        def _():
            fire(k0 + 2, set0, sa)
        compute(k0 + 1, set1)
        return cy

    lax.fori_loop(0, NCH1 // 2, pipe, 0)
    fire(NCH1 - 1, set0, sa)
    drain(NCH1 - 1, set0, sa)
    compute(NCH1 - 1, set0)

    def sig(g, cy):
        sl = pl.ds(g * 16, 16)
        v = dots[sl]
        dots[sl] = 1.0 / (1.0 + jnp.exp(-(v + b2)))
        return cy
    lax.fori_loop(0, (EPW + 8) // 16, sig, 0)
    pltpu.sync_copy(dots.at[pl.ds(0, EPW)], alpha_out.at[pl.ds(base0, EPW)])


def _alpha_pass(ps, pr, pq, sub, rel, ridx, qrel, wcat):
    mesh = plsc.VectorSubcoreMesh(core_axis_name="c", subcore_axis_name="s")
    f = pl.kernel(
        _alpha_body,
        out_type=jax.ShapeDtypeStruct((N_EDGES,), jnp.float32),
        mesh=mesh,
        scratch_types=[
            pltpu.VMEM((EPW,), jnp.int32),
            pltpu.VMEM((EPW,), jnp.int32),
            pltpu.VMEM((EPW + 8,), jnp.int32),
            pltpu.VMEM((EPW + 8,), jnp.int32),
            pltpu.VMEM((N_QUERIES,), jnp.int32),
            pltpu.VMEM((B1, D), jnp.float32),
            pltpu.VMEM((B1, D), jnp.float32),
            pltpu.VMEM((B1, D), jnp.float32),
            pltpu.VMEM((B1, D), jnp.float32),
            pltpu.VMEM((B1, D), jnp.float32),
            pltpu.VMEM((B1, D), jnp.float32),
            pltpu.VMEM((272,), jnp.float32),
            pltpu.VMEM((EPW + 8,), jnp.float32),
            pltpu.SemaphoreType.DMA,
            pltpu.SemaphoreType.DMA,
        ],
        compiler_params=pltpu.CompilerParams(needs_layout_passes=False, use_tc_tiling_on_sc=False),
    )
    return f(ps, pr, pq, sub, rel, ridx, qrel, wcat)


def _agg_body(ls, lr, sub, rel, obj, alpha, out,
              shared, isub, irel, iobj, abuf, ga, gb, gc, gd, obuf,
              sa, sb, ss0, ss1):
    c = lax.axis_index("c")
    s = lax.axis_index("s")
    coff = c * 128
    roff = c * MP

    def zrow(i, cy):
        for j in range(8):
            obuf[i, pl.ds(j * 16, 16)] = jnp.zeros((16,), jnp.float32)
        return cy
    lax.fori_loop(0, RC, zrow, 0)

    def zcopy(t, cy):
        pltpu.sync_copy(obuf, shared.at[pl.ds(s * RPT + t * RC, RC)])
        return cy
    lax.fori_loop(0, NRC, zcopy, 0)
    plsc.subcore_barrier()

    base0 = s * EPT

    def load_window(w):
        wb = base0 + w * WIN
        pltpu.sync_copy(sub.at[pl.ds(wb, WIN)], isub)
        pltpu.sync_copy(rel.at[pl.ds(wb, WIN)], irel)
        pltpu.sync_copy(obj.at[pl.ds(wb, WIN)], iobj)
        pltpu.sync_copy(alpha.at[pl.ds(wb, WIN)], abuf.at[pl.ds(0, WIN)])

        def offs(i, cy):
            sl = pl.ds(i * 16, 16)
            isub[sl] = isub[sl] + roff
            irel[sl] = irel[sl] + roff
            return cy
        lax.fori_loop(0, WIN // 16, offs, 0)

    def fire(kk, bufs, sem):
        st = kk * B2
        ga_, gb_ = bufs
        pltpu.async_copy(ls.at[isub.at[pl.ds(st, B2)]], ga_, sem)
        pltpu.async_copy(lr.at[irel.at[pl.ds(st, B2)]], gb_, sem)

    def drain(kk, bufs, sem):
        st = kk * B2
        ga_, gb_ = bufs
        pltpu.make_async_copy(ls.at[isub.at[pl.ds(st, B2)]], ga_, sem).wait()
        pltpu.make_async_copy(lr.at[irel.at[pl.ds(st, B2)]], gb_, sem).wait()

    def work(kk, bufs):
        st = kk * B2
        ga_, gb_ = bufs

        def group_body(g, cy):
            av = abuf[pl.ds(st + g * 16, 16)]
            for l in range(16):
                e = g * 16 + l
                a = av[l]
                for j in range(8):
                    sl = pl.ds(j * 16, 16)
                    ga_[e, sl] = (ga_[e, sl] + gb_[e, sl]) * a
            return cy

        lax.fori_loop(0, B2 // 16, group_body, 0)
        av = abuf[pl.ds(st + 32, 16)]
        for l in range(8):
            e = 32 + l
            a = av[l]
            for j in range(8):
                sl = pl.ds(j * 16, 16)
                ga_[e, sl] = (ga_[e, sl] + gb_[e, sl]) * a

    def scat_fire(kk, bufs, sem):
        ga_, _ = bufs
        pltpu.async_copy(ga_, shared.at[iobj.at[pl.ds(kk * B2, B2)]], sem,
                         add=True)

    def scat_drain(kk, bufs, sem):
        ga_, _ = bufs
        pltpu.make_async_copy(ga_, shared.at[iobj.at[pl.ds(kk * B2, B2)]],
                              sem).wait()

    set0 = (ga, gb)
    set1 = (gc, gd)

    def window_body(w, wcy):
        load_window(w)
        fire(0, set0, sa)

        def pipe(k, cy):
            even = lax.rem(k, 2) == 0

            @pl.when(even)
            def _():
                drain(k, set0, sa)

                @pl.when(k < CPW - 1)
                def _():
                    fire(k + 1, set1, sb)

                @pl.when(k >= 2)
                def _():
                    scat_drain(k - 2, set0, ss0)
                work(k, set0)
                scat_fire(k, set0, ss0)

            @pl.when(jnp.logical_not(even))
            def _():
                drain(k, set1, sb)

                @pl.when(k < CPW - 1)
                def _():
                    fire(k + 1, set0, sa)

                @pl.when(k >= 2)
                def _():
                    scat_drain(k - 2, set1, ss1)
                work(k, set1)
                scat_fire(k, set1, ss1)
            return cy

        lax.fori_loop(0, CPW, pipe, 0)
        scat_drain(CPW - 2, set0, ss0)
        scat_drain(CPW - 1, set1, ss1)
        return wcy

    lax.fori_loop(0, NWIN, window_body, 0)
    plsc.subcore_barrier()

    def ocopy(t, cy):
        r0 = s * RPT + t * RC
        pltpu.sync_copy(shared.at[pl.ds(r0, RC)], obuf)

        def rrow(i, icy):
            for j in range(8):
                sl = pl.ds(j * 16, 16)
                obuf[i, sl] = jnp.maximum(obuf[i, sl], 0.0)
            return icy
        lax.fori_loop(0, RC, rrow, 0)
        pltpu.sync_copy(obuf, out.at[pl.ds(r0, RC), pl.ds(coff, 128)])
        return cy
    lax.fori_loop(0, NRC, ocopy, 0)


def _agg_pass(ls, lr, sub, rel, obj, alpha):
    mesh = plsc.VectorSubcoreMesh(core_axis_name="c", subcore_axis_name="s")
    f = pl.kernel(
        _agg_body,
        out_type=jax.ShapeDtypeStruct((N_NODES, D), jnp.float32),
        mesh=mesh,
        scratch_types=[
            pltpu.VMEM_SHARED((N_NODES, 128), jnp.float32),
            pltpu.VMEM((WIN,), jnp.int32),
            pltpu.VMEM((WIN,), jnp.int32),
            pltpu.VMEM((WIN,), jnp.int32),
            pltpu.VMEM((WIN + 16,), jnp.float32),
            pltpu.VMEM((B2, 128), jnp.float32),
            pltpu.VMEM((B2, 128), jnp.float32),
            pltpu.VMEM((B2, 128), jnp.float32),
            pltpu.VMEM((B2, 128), jnp.float32),
            pltpu.VMEM((RC, 128), jnp.float32),
            pltpu.SemaphoreType.DMA,
            pltpu.SemaphoreType.DMA,
            pltpu.SemaphoreType.DMA,
            pltpu.SemaphoreType.DMA,
        ],
        compiler_params=pltpu.CompilerParams(needs_layout_passes=False, use_tc_tiling_on_sc=False),
    )
    return f(ls, lr, sub, rel, obj, alpha)


def kernel(q_rel, layer_input, edges, nodes, n_ent, rela_embed,
           Ws_attn, Wr_attn, Wqr_attn_W, Wqr_attn_b,
           w_alpha_W, w_alpha_b, W_h):
    sub = edges[:, 4]
    rel = edges[:, 2]
    obj = edges[:, 5]
    ridx = edges[:, 0]

    xl = jnp.pad(layer_input, ((0, MP - layer_input.shape[0]), (0, 0)))
    xr = jnp.pad(rela_embed, ((0, MP - rela_embed.shape[0]), (0, 0)))

    ps = _mm(xl, Ws_attn)
    pr = _mm(xr, Wr_attn)
    pq = _mm_bias(xr, Wqr_attn_W, Wqr_attn_b)
    ls = _mm_colsplit(xl, W_h)
    lr = _mm_colsplit(xr, W_h)

    wcat = jnp.concatenate(
        [w_alpha_W[:, 0], w_alpha_b, jnp.zeros((15,), jnp.float32)])

    alpha = _alpha_pass(ps, pr, pq, sub, rel, ridx, q_rel, wcat)
    return _agg_pass(ls, lr, sub, rel, obj, alpha)


# speedup vs baseline: 4.0420x; 1.1004x over previous
import functools

import jax
import jax.numpy as jnp
import numpy as np
from jax import lax
from jax.experimental import pallas as pl
from jax.experimental.pallas import tpu as pltpu
from jax.experimental.pallas import tpu_sc as plsc

N_NODES = 10000
N_QUERIES = 10000
N_EDGES = 160000
D = 256
MP = 10240
NC, NS = 2, 16
NW = NC * NS

EPW = N_EDGES // NW
B1 = 40
NCH1 = EPW // B1

EPT = N_EDGES // NS
B2 = 40
WIN = 2000
NWIN = EPT // WIN
CPW = WIN // B2
RPT = N_NODES // NS
RC = 25
NRC = RPT // RC

_PERM = np.empty((D,), np.int32)
for _g in range(D // 32):
    for _t in range(16):
        _PERM[32 * _g + 2 * _t] = 32 * _g + _t
        _PERM[32 * _g + 2 * _t + 1] = 32 * _g + 16 + _t
_EO = np.empty((D,), np.int32)
for _g in range(D // 32):
    _EO[32 * _g:32 * _g + 16] = np.arange(32 * _g, 32 * _g + 32, 2)
    _EO[32 * _g + 16:32 * _g + 32] = np.arange(32 * _g + 1, 32 * _g + 32, 2)


def _mm_kernel(x_ref, w_ref, o_ref):
    o_ref[...] = jnp.dot(x_ref[...], w_ref[...],
                         preferred_element_type=jnp.float32
                         ).astype(o_ref.dtype)


def _mm_bias_kernel(x_ref, w_ref, b_ref, o_ref):
    o_ref[...] = (jnp.dot(x_ref[...], w_ref[...],
                          preferred_element_type=jnp.float32)
                  + b_ref[0:1, :]).astype(o_ref.dtype)


def _mm(x, w, out_dtype=jnp.float32):
    m, k = x.shape
    n = w.shape[1]
    bm = 1024
    return pl.pallas_call(
        _mm_kernel,
        grid=(m // bm,),
        in_specs=[pl.BlockSpec((bm, k), lambda i: (i, 0)),
                  pl.BlockSpec((k, n), lambda i: (0, 0))],
        out_specs=pl.BlockSpec((bm, n), lambda i: (i, 0)),
        out_shape=jax.ShapeDtypeStruct((m, n), out_dtype),
    )(x, w)


def _mm_bias(x, w, b, out_dtype=jnp.float32):
    m, k = x.shape
    n = w.shape[1]
    bm = 1024
    b8 = jnp.zeros((8, n), jnp.float32).at[0].set(b)
    return pl.pallas_call(
        _mm_bias_kernel,
        grid=(m // bm,),
        in_specs=[pl.BlockSpec((bm, k), lambda i: (i, 0)),
                  pl.BlockSpec((k, n), lambda i: (0, 0)),
                  pl.BlockSpec((8, n), lambda i: (0, 0))],
        out_specs=pl.BlockSpec((bm, n), lambda i: (i, 0)),
        out_shape=jax.ShapeDtypeStruct((m, n), out_dtype),
    )(x, w, b8)


def _mm_colsplit(x, w, out_dtype=jnp.float32):
    m, k = x.shape
    bm = 1024
    nb = m // bm
    return pl.pallas_call(
        _mm_kernel,
        grid=(nb, 2),
        in_specs=[pl.BlockSpec((bm, k), lambda i, j: (i, 0)),
                  pl.BlockSpec((k, 128), lambda i, j: (0, j))],
        out_specs=pl.BlockSpec((bm, 128), lambda i, j: (j * nb + i, 0)),
        out_shape=jax.ShapeDtypeStruct((2 * m, 128), out_dtype),
    )(x, w)


def _alpha_body(ps, pr, pq, sub, rel, ridx, qrel, wcat, alpha_out,
                isub, irel, irx, iqr, qtab, ra0, rb0, rc0, ra1, rb1, rc1,
                wv, dots, sa, sb):
    c = lax.axis_index("c")
    s = lax.axis_index("s")
    wid = s * NC + c
    base0 = wid * EPW
    pltpu.sync_copy(wcat, wv)
    b2 = wv[pl.ds(256, 16)][0]
    wregs = [wv[pl.ds(j * 16, 16)] for j in range(16)]
    lane = lax.iota(jnp.int32, 16)
    m15 = lane == 15

    irx[pl.ds(EPW - 8, 16)] = jnp.zeros((16,), jnp.int32)
    pltpu.sync_copy(sub.at[pl.ds(base0, EPW)], isub)
    pltpu.sync_copy(rel.at[pl.ds(base0, EPW)], irel)
    pltpu.sync_copy(ridx.at[pl.ds(base0, EPW)], irx.at[pl.ds(0, EPW)])
    pltpu.sync_copy(qrel, qtab)

    def qcomp(g, cy):
        sl = pl.ds(g * 16, 16)
        iqr[sl] = plsc.load_gather(qtab, [irx[sl]])
        return cy
    lax.fori_loop(0, (EPW + 8) // 16, qcomp, 0)

    def fire(kk, bufs, sem):
        st = kk * B1
        ra, rb, rc_ = bufs
        pltpu.async_copy(ps.at[isub.at[pl.ds(st, B1)]], ra, sem)
        pltpu.async_copy(pr.at[irel.at[pl.ds(st, B1)]], rb, sem)
        pltpu.async_copy(pq.at[iqr.at[pl.ds(st, B1)]], rc_, sem)

    def drain(kk, bufs, sem):
        st = kk * B1
        ra, rb, rc_ = bufs
        pltpu.make_async_copy(ps.at[isub.at[pl.ds(st, B1)]], ra, sem).wait()
        pltpu.make_async_copy(pr.at[irel.at[pl.ds(st, B1)]], rb, sem).wait()
        pltpu.make_async_copy(pq.at[iqr.at[pl.ds(st, B1)]], rc_, sem).wait()

    def compute(kk, bufs):
        ra, rb, rc_ = bufs

        def edge_body(e, cy):
            acc = jnp.zeros((16,), jnp.float32)
            for g in range(16 // 2):
                sl = pl.ds(g * 32, 32)
                s32 = ra[e, sl] + rb[e, sl] + rc_[e, sl]
                ve, vo = plsc.unpack(s32, format=plsc.PackFormat.INTERLEAVED)
                acc = acc + jnp.maximum(ve, 0.0) * wregs[2 * g]
                acc = acc + jnp.maximum(vo, 0.0) * wregs[2 * g + 1]
            tot = plsc.cumsum(acc)
            plsc.store_scatter(dots, [jnp.full((16,), kk * B1 + e, jnp.int32)],
                               tot, mask=m15)
            return cy

        lax.fori_loop(0, B1, edge_body, 0)

    set0 = (ra0, rb0, rc0)
    set1 = (ra1, rb1, rc1)
    fire(0, set0, sa)

    def pipe(g, cy):
        k0 = g * 2
        drain(k0, set0, sa)
        fire(k0 + 1, set1, sb)
        compute(k0, set0)
        drain(k0 + 1, set1, sb)

        @pl.when(g < NCH1 // 2 - 1)
        def _():
            fire(k0 + 2, set0, sa)
        compute(k0 + 1, set1)
        return cy

    lax.fori_loop(0, NCH1 // 2, pipe, 0)
    fire(NCH1 - 1, set0, sa)
    drain(NCH1 - 1, set0, sa)
    compute(NCH1 - 1, set0)

    def sig(g, cy):
        sl = pl.ds(g * 16, 16)
        v = dots[sl]
        dots[sl] = 1.0 / (1.0 + jnp.exp(-(v + b2)))
        return cy
    lax.fori_loop(0, (EPW + 8) // 16, sig, 0)
    pltpu.sync_copy(dots.at[pl.ds(0, EPW)], alpha_out.at[pl.ds(base0, EPW)])


def _alpha_pass(ps, pr, pq, sub, rel, ridx, qrel, wcat):
    mesh = plsc.VectorSubcoreMesh(core_axis_name="c", subcore_axis_name="s")
    f = pl.kernel(
        _alpha_body,
        out_type=jax.ShapeDtypeStruct((N_EDGES,), jnp.float32),
        mesh=mesh,
        scratch_types=[
            pltpu.VMEM((EPW,), jnp.int32),
            pltpu.VMEM((EPW,), jnp.int32),
            pltpu.VMEM((EPW + 8,), jnp.int32),
            pltpu.VMEM((EPW + 8,), jnp.int32),
            pltpu.VMEM((N_QUERIES,), jnp.int32),
            pltpu.VMEM((B1, D), jnp.bfloat16),
            pltpu.VMEM((B1, D), jnp.bfloat16),
            pltpu.VMEM((B1, D), jnp.bfloat16),
            pltpu.VMEM((B1, D), jnp.bfloat16),
            pltpu.VMEM((B1, D), jnp.bfloat16),
            pltpu.VMEM((B1, D), jnp.bfloat16),
            pltpu.VMEM((272,), jnp.float32),
            pltpu.VMEM((EPW + 8,), jnp.float32),
            pltpu.SemaphoreType.DMA,
            pltpu.SemaphoreType.DMA,
        ],
        compiler_params=pltpu.CompilerParams(needs_layout_passes=False, use_tc_tiling_on_sc=False),
    )
    return f(ps, pr, pq, sub, rel, ridx, qrel, wcat)


def _agg_body(ls, lr, sub, rel, obj, alpha, out,
              shared, isub, irel, iobj, abuf, ga, gb, gc, gd, mb0, mb1, obuf,
              sa, sb, ss0, ss1):
    c = lax.axis_index("c")
    s = lax.axis_index("s")
    coff = c * 128
    roff = c * MP

    def zrow(i, cy):
        for j in range(8):
            obuf[i, pl.ds(j * 16, 16)] = jnp.zeros((16,), jnp.float32)
        return cy
    lax.fori_loop(0, RC, zrow, 0)

    def zcopy(t, cy):
        pltpu.sync_copy(obuf, shared.at[pl.ds(s * RPT + t * RC, RC)])
        return cy
    lax.fori_loop(0, NRC, zcopy, 0)
    plsc.subcore_barrier()

    base0 = s * EPT

    def load_window(w):
        wb = base0 + w * WIN
        pltpu.sync_copy(sub.at[pl.ds(wb, WIN)], isub)
        pltpu.sync_copy(rel.at[pl.ds(wb, WIN)], irel)
        pltpu.sync_copy(obj.at[pl.ds(wb, WIN)], iobj)
        pltpu.sync_copy(alpha.at[pl.ds(wb, WIN)], abuf.at[pl.ds(0, WIN)])

        def offs(i, cy):
            sl = pl.ds(i * 16, 16)
            isub[sl] = isub[sl] + roff
            irel[sl] = irel[sl] + roff
            return cy
        lax.fori_loop(0, WIN // 16, offs, 0)

    def fire(kk, bufs, sem):
        st = kk * B2
        ga_, gb_ = bufs
        pltpu.async_copy(ls.at[isub.at[pl.ds(st, B2)]], ga_, sem)
        pltpu.async_copy(lr.at[irel.at[pl.ds(st, B2)]], gb_, sem)

    def drain(kk, bufs, sem):
        st = kk * B2
        ga_, gb_ = bufs
        pltpu.make_async_copy(ls.at[isub.at[pl.ds(st, B2)]], ga_, sem).wait()
        pltpu.make_async_copy(lr.at[irel.at[pl.ds(st, B2)]], gb_, sem).wait()

    def work(kk, bufs, mb):
        st = kk * B2
        ga_, gb_ = bufs

        def do_edge(e, a):
            for g in range(4):
                sl = pl.ds(g * 32, 32)
                s32 = ga_[e, sl] + gb_[e, sl]
                ve, vo = plsc.unpack(s32, format=plsc.PackFormat.INTERLEAVED)
                mb[e, pl.ds(g * 32, 16)] = ve * a
                mb[e, pl.ds(g * 32 + 16, 16)] = vo * a

        def group_body(g, cy):
            av = abuf[pl.ds(st + g * 16, 16)]
            for l in range(16):
                do_edge(g * 16 + l, av[l])
            return cy

        lax.fori_loop(0, B2 // 16, group_body, 0)
        av = abuf[pl.ds(st + 32, 16)]
        for l in range(8):
            do_edge(32 + l, av[l])

    def scat_fire(kk, mb, sem):
        pltpu.async_copy(mb, shared.at[iobj.at[pl.ds(kk * B2, B2)]], sem,
                         add=True)

    def scat_drain(kk, mb, sem):
        pltpu.make_async_copy(mb, shared.at[iobj.at[pl.ds(kk * B2, B2)]],
                              sem).wait()

    set0 = (ga, gb)
    set1 = (gc, gd)

    def window_body(w, wcy):
        load_window(w)
        fire(0, set0, sa)

        def pipe(k, cy):
            even = lax.rem(k, 2) == 0

            @pl.when(even)
            def _():
                drain(k, set0, sa)

                @pl.when(k < CPW - 1)
                def _():
                    fire(k + 1, set1, sb)

                @pl.when(k >= 2)
                def _():
                    scat_drain(k - 2, mb0, ss0)
                work(k, set0, mb0)
                scat_fire(k, mb0, ss0)

            @pl.when(jnp.logical_not(even))
            def _():
                drain(k, set1, sb)

                @pl.when(k < CPW - 1)
                def _():
                    fire(k + 1, set0, sa)

                @pl.when(k >= 2)
                def _():
                    scat_drain(k - 2, mb1, ss1)
                work(k, set1, mb1)
                scat_fire(k, mb1, ss1)
            return cy

        lax.fori_loop(0, CPW, pipe, 0)
        scat_drain(CPW - 2, mb0, ss0)
        scat_drain(CPW - 1, mb1, ss1)
        return wcy

    lax.fori_loop(0, NWIN, window_body, 0)
    plsc.subcore_barrier()

    def ocopy(t, cy):
        r0 = s * RPT + t * RC
        pltpu.sync_copy(shared.at[pl.ds(r0, RC)], obuf)

        def rrow(i, icy):
            for j in range(8):
                sl = pl.ds(j * 16, 16)
                obuf[i, sl] = jnp.maximum(obuf[i, sl], 0.0)
            return icy
        lax.fori_loop(0, RC, rrow, 0)
        pltpu.sync_copy(obuf, out.at[pl.ds(r0, RC), pl.ds(coff, 128)])
        return cy
    lax.fori_loop(0, NRC, ocopy, 0)


def _agg_pass(ls, lr, sub, rel, obj, alpha):
    mesh = plsc.VectorSubcoreMesh(core_axis_name="c", subcore_axis_name="s")
    f = pl.kernel(
        _agg_body,
        out_type=jax.ShapeDtypeStruct((N_NODES, D), jnp.float32),
        mesh=mesh,
        scratch_types=[
            pltpu.VMEM_SHARED((N_NODES, 128), jnp.float32),
            pltpu.VMEM((WIN,), jnp.int32),
            pltpu.VMEM((WIN,), jnp.int32),
            pltpu.VMEM((WIN,), jnp.int32),
            pltpu.VMEM((WIN + 16,), jnp.float32),
            pltpu.VMEM((B2, 128), jnp.bfloat16),
            pltpu.VMEM((B2, 128), jnp.bfloat16),
            pltpu.VMEM((B2, 128), jnp.bfloat16),
            pltpu.VMEM((B2, 128), jnp.bfloat16),
            pltpu.VMEM((B2, 128), jnp.float32),
            pltpu.VMEM((B2, 128), jnp.float32),
            pltpu.VMEM((RC, 128), jnp.float32),
            pltpu.SemaphoreType.DMA,
            pltpu.SemaphoreType.DMA,
            pltpu.SemaphoreType.DMA,
            pltpu.SemaphoreType.DMA,
        ],
        compiler_params=pltpu.CompilerParams(needs_layout_passes=False, use_tc_tiling_on_sc=False),
    )
    return f(ls, lr, sub, rel, obj, alpha)


def kernel(q_rel, layer_input, edges, nodes, n_ent, rela_embed,
           Ws_attn, Wr_attn, Wqr_attn_W, Wqr_attn_b,
           w_alpha_W, w_alpha_b, W_h):
    sub = edges[:, 4]
    rel = edges[:, 2]
    obj = edges[:, 5]
    ridx = edges[:, 0]

    xl = jnp.pad(layer_input, ((0, MP - layer_input.shape[0]), (0, 0)))
    xr = jnp.pad(rela_embed, ((0, MP - rela_embed.shape[0]), (0, 0)))

    bf = jnp.bfloat16
    ps = _mm(xl, Ws_attn, bf)
    pr = _mm(xr, Wr_attn, bf)
    pq = _mm_bias(xr, Wqr_attn_W, Wqr_attn_b, bf)
    whp = W_h[:, _PERM]
    ls = _mm_colsplit(xl, whp, bf)
    lr = _mm_colsplit(xr, whp, bf)

    wcat = jnp.concatenate(
        [w_alpha_W[_EO, 0], w_alpha_b, jnp.zeros((15,), jnp.float32)])

    alpha = _alpha_pass(ps, pr, pq, sub, rel, ridx, q_rel, wcat)
    return _agg_pass(ls, lr, sub, rel, obj, alpha)


# speedup vs baseline: 4.3657x; 1.0801x over previous
import functools

import jax
import jax.numpy as jnp
import numpy as np
from jax import lax
from jax.experimental import pallas as pl
from jax.experimental.pallas import tpu as pltpu
from jax.experimental.pallas import tpu_sc as plsc

N_NODES = 10000
N_QUERIES = 10000
N_EDGES = 160000
D = 256
MP = 10240
NC, NS = 2, 16
NW = NC * NS

EPW = N_EDGES // NW
B1 = 40
NCH1 = EPW // B1

EPT = N_EDGES // NS
B2 = 40
WIN = 2000
NWIN = EPT // WIN
CPW = WIN // B2
RPT = N_NODES // NS
RC = 25
NRC = RPT // RC

_PERM = np.empty((D,), np.int32)
for _g in range(D // 32):
    for _t in range(16):
        _PERM[32 * _g + 2 * _t] = 32 * _g + _t
        _PERM[32 * _g + 2 * _t + 1] = 32 * _g + 16 + _t
_EO = np.empty((D,), np.int32)
for _g in range(D // 32):
    _EO[32 * _g:32 * _g + 16] = np.arange(32 * _g, 32 * _g + 32, 2)
    _EO[32 * _g + 16:32 * _g + 32] = np.arange(32 * _g + 1, 32 * _g + 32, 2)


def _mm_kernel(x_ref, w_ref, o_ref):
    o_ref[...] = jnp.dot(x_ref[...], w_ref[...],
                         preferred_element_type=jnp.float32
                         ).astype(o_ref.dtype)


def _mm_bias_kernel(x_ref, w_ref, b_ref, o_ref):
    o_ref[...] = (jnp.dot(x_ref[...], w_ref[...],
                          preferred_element_type=jnp.float32)
                  + b_ref[0:1, :]).astype(o_ref.dtype)


def _mm(x, w, out_dtype=jnp.float32):
    m, k = x.shape
    n = w.shape[1]
    bm = 1024
    return pl.pallas_call(
        _mm_kernel,
        grid=(m // bm,),
        in_specs=[pl.BlockSpec((bm, k), lambda i: (i, 0)),
                  pl.BlockSpec((k, n), lambda i: (0, 0))],
        out_specs=pl.BlockSpec((bm, n), lambda i: (i, 0)),
        out_shape=jax.ShapeDtypeStruct((m, n), out_dtype),
    )(x, w)


def _mm_bias(x, w, b, out_dtype=jnp.float32):
    m, k = x.shape
    n = w.shape[1]
    bm = 1024
    b8 = jnp.zeros((8, n), jnp.float32).at[0].set(b)
    return pl.pallas_call(
        _mm_bias_kernel,
        grid=(m // bm,),
        in_specs=[pl.BlockSpec((bm, k), lambda i: (i, 0)),
                  pl.BlockSpec((k, n), lambda i: (0, 0)),
                  pl.BlockSpec((8, n), lambda i: (0, 0))],
        out_specs=pl.BlockSpec((bm, n), lambda i: (i, 0)),
        out_shape=jax.ShapeDtypeStruct((m, n), out_dtype),
    )(x, w, b8)


def _mm_colsplit(x, w, out_dtype=jnp.float32):
    m, k = x.shape
    bm = 1024
    nb = m // bm
    return pl.pallas_call(
        _mm_kernel,
        grid=(nb, 2),
        in_specs=[pl.BlockSpec((bm, k), lambda i, j: (i, 0)),
                  pl.BlockSpec((k, 128), lambda i, j: (0, j))],
        out_specs=pl.BlockSpec((bm, 128), lambda i, j: (j * nb + i, 0)),
        out_shape=jax.ShapeDtypeStruct((2 * m, 128), out_dtype),
    )(x, w)


def _alpha_body(ps, pr, pq, sub, rel, ridx, qrel, wcat, alpha_out,
                isub, irel, irx, iqr, qtab, ra0, rb0, rc0, ra1, rb1, rc1,
                wv, dots, sa, sb):
    c = lax.axis_index("c")
    s = lax.axis_index("s")
    wid = s * NC + c
    base0 = wid * EPW
    pltpu.sync_copy(wcat, wv)
    b2 = wv[pl.ds(256, 16)][0]
    wregs = [wv[pl.ds(j * 16, 16)] for j in range(16)]
    lane = lax.iota(jnp.int32, 16)
    m15 = lane == 15

    irx[pl.ds(EPW - 8, 16)] = jnp.zeros((16,), jnp.int32)
    pltpu.sync_copy(sub.at[pl.ds(base0, EPW)], isub)
    pltpu.sync_copy(rel.at[pl.ds(base0, EPW)], irel)
    pltpu.sync_copy(ridx.at[pl.ds(base0, EPW)], irx.at[pl.ds(0, EPW)])
    pltpu.sync_copy(qrel, qtab)

    def qcomp(g, cy):
        sl = pl.ds(g * 16, 16)
        iqr[sl] = plsc.load_gather(qtab, [irx[sl]])
        return cy
    lax.fori_loop(0, (EPW + 8) // 16, qcomp, 0)

    def fire(kk, bufs, sem):
        st = kk * B1
        ra, rb, rc_ = bufs
        pltpu.async_copy(ps.at[isub.at[pl.ds(st, B1)]], ra, sem)
        pltpu.async_copy(pr.at[irel.at[pl.ds(st, B1)]], rb, sem)
        pltpu.async_copy(pq.at[iqr.at[pl.ds(st, B1)]], rc_, sem)

    def drain(kk, bufs, sem):
        st = kk * B1
        ra, rb, rc_ = bufs
        pltpu.make_async_copy(ps.at[isub.at[pl.ds(st, B1)]], ra, sem).wait()
        pltpu.make_async_copy(pr.at[irel.at[pl.ds(st, B1)]], rb, sem).wait()
        pltpu.make_async_copy(pq.at[iqr.at[pl.ds(st, B1)]], rc_, sem).wait()

    def compute(kk, bufs):
        ra, rb, rc_ = bufs

        def edge_body(e, cy):
            acc = jnp.zeros((16,), jnp.float32)
            for g in range(16 // 2):
                sl = pl.ds(g * 32, 32)
                s32 = ra[e, sl] + rb[e, sl] + rc_[e, sl]
                ve, vo = plsc.unpack(s32, format=plsc.PackFormat.INTERLEAVED)
                acc = acc + jnp.maximum(ve, 0.0) * wregs[2 * g]
                acc = acc + jnp.maximum(vo, 0.0) * wregs[2 * g + 1]
            tot = plsc.cumsum(acc)
            plsc.store_scatter(dots, [jnp.full((16,), kk * B1 + e, jnp.int32)],
                               tot, mask=m15)
            return cy

        lax.fori_loop(0, B1, edge_body, 0)

    set0 = (ra0, rb0, rc0)
    set1 = (ra1, rb1, rc1)
    fire(0, set0, sa)

    def pipe(g, cy):
        k0 = g * 2
        drain(k0, set0, sa)
        fire(k0 + 1, set1, sb)
        compute(k0, set0)
        drain(k0 + 1, set1, sb)

        @pl.when(g < NCH1 // 2 - 1)
        def _():
            fire(k0 + 2, set0, sa)
        compute(k0 + 1, set1)
        return cy

    lax.fori_loop(0, NCH1 // 2, pipe, 0)
    fire(NCH1 - 1, set0, sa)
    drain(NCH1 - 1, set0, sa)
    compute(NCH1 - 1, set0)

    def sig(g, cy):
        sl = pl.ds(g * 16, 16)
        v = dots[sl]
        dots[sl] = 1.0 / (1.0 + jnp.exp(-(v + b2)))
        return cy
    lax.fori_loop(0, (EPW + 8) // 16, sig, 0)
    pltpu.sync_copy(dots.at[pl.ds(0, EPW)], alpha_out.at[pl.ds(base0, EPW)])


def _alpha_pass(ps, pr, pq, sub, rel, ridx, qrel, wcat):
    mesh = plsc.VectorSubcoreMesh(core_axis_name="c", subcore_axis_name="s")
    f = pl.kernel(
        _alpha_body,
        out_type=jax.ShapeDtypeStruct((N_EDGES,), jnp.float32),
        mesh=mesh,
        scratch_types=[
            pltpu.VMEM((EPW,), jnp.int32),
            pltpu.VMEM((EPW,), jnp.int32),
            pltpu.VMEM((EPW + 8,), jnp.int32),
            pltpu.VMEM((EPW + 8,), jnp.int32),
            pltpu.VMEM((N_QUERIES,), jnp.int32),
            pltpu.VMEM((B1, D), jnp.bfloat16),
            pltpu.VMEM((B1, D), jnp.bfloat16),
            pltpu.VMEM((B1, D), jnp.bfloat16),
            pltpu.VMEM((B1, D), jnp.bfloat16),
            pltpu.VMEM((B1, D), jnp.bfloat16),
            pltpu.VMEM((B1, D), jnp.bfloat16),
            pltpu.VMEM((272,), jnp.float32),
            pltpu.VMEM((EPW + 8,), jnp.float32),
            pltpu.SemaphoreType.DMA,
            pltpu.SemaphoreType.DMA,
        ],
        compiler_params=pltpu.CompilerParams(needs_layout_passes=False, use_tc_tiling_on_sc=False),
    )
    return f(ps, pr, pq, sub, rel, ridx, qrel, wcat)


def _agg_body(ls, lr, sub, rel, obj, alpha, out,
              shared, isub, irel, iobj, abuf, ga, gb, gc, gd, mb0, mb1,
              obuf, sbuf, sa, sb, ss0, ss1):
    c = lax.axis_index("c")
    s = lax.axis_index("s")
    coff = c * 128
    roff = c * MP

    def zrow(i, cy):
        for j in range(4):
            sbuf[i, pl.ds(j * 32, 32)] = jnp.zeros((32,), jnp.bfloat16)
        return cy
    lax.fori_loop(0, RC, zrow, 0)

    def zcopy(t, cy):
        pltpu.sync_copy(sbuf, shared.at[pl.ds(s * RPT + t * RC, RC)])
        return cy
    lax.fori_loop(0, NRC, zcopy, 0)
    plsc.subcore_barrier()

    base0 = s * EPT

    def load_window(w):
        wb = base0 + w * WIN
        pltpu.sync_copy(sub.at[pl.ds(wb, WIN)], isub)
        pltpu.sync_copy(rel.at[pl.ds(wb, WIN)], irel)
        pltpu.sync_copy(obj.at[pl.ds(wb, WIN)], iobj)
        pltpu.sync_copy(alpha.at[pl.ds(wb, WIN)], abuf.at[pl.ds(0, WIN)])

        def offs(i, cy):
            sl = pl.ds(i * 16, 16)
            isub[sl] = isub[sl] + roff
            irel[sl] = irel[sl] + roff
            return cy
        lax.fori_loop(0, WIN // 16, offs, 0)

    def fire(kk, bufs, sem):
        st = kk * B2
        ga_, gb_ = bufs
        pltpu.async_copy(ls.at[isub.at[pl.ds(st, B2)]], ga_, sem)
        pltpu.async_copy(lr.at[irel.at[pl.ds(st, B2)]], gb_, sem)

    def drain(kk, bufs, sem):
        st = kk * B2
        ga_, gb_ = bufs
        pltpu.make_async_copy(ls.at[isub.at[pl.ds(st, B2)]], ga_, sem).wait()
        pltpu.make_async_copy(lr.at[irel.at[pl.ds(st, B2)]], gb_, sem).wait()

    def work(kk, bufs, mb):
        st = kk * B2
        ga_, gb_ = bufs

        def do_edge(e, a):
            for g in range(4):
                sl = pl.ds(g * 32, 32)
                s32 = ga_[e, sl] + gb_[e, sl]
                ve, vo = plsc.unpack(s32, format=plsc.PackFormat.INTERLEAVED)
                mb[e, sl] = plsc.pack(ve * a, vo * a,
                                      format=plsc.PackFormat.INTERLEAVED)

        def group_body(g, cy):
            av = abuf[pl.ds(st + g * 16, 16)]
            for l in range(16):
                do_edge(g * 16 + l, av[l])
            return cy

        lax.fori_loop(0, B2 // 16, group_body, 0)
        av = abuf[pl.ds(st + 32, 16)]
        for l in range(8):
            do_edge(32 + l, av[l])

    def scat_fire(kk, mb, sem):
        pltpu.async_copy(mb, shared.at[iobj.at[pl.ds(kk * B2, B2)]], sem,
                         add=True)

    def scat_drain(kk, mb, sem):
        pltpu.make_async_copy(mb, shared.at[iobj.at[pl.ds(kk * B2, B2)]],
                              sem).wait()

    set0 = (ga, gb)
    set1 = (gc, gd)

    def window_body(w, wcy):
        load_window(w)
        fire(0, set0, sa)

        def pipe(k, cy):
            even = lax.rem(k, 2) == 0

            @pl.when(even)
            def _():
                drain(k, set0, sa)

                @pl.when(k < CPW - 1)
                def _():
                    fire(k + 1, set1, sb)

                @pl.when(k >= 2)
                def _():
                    scat_drain(k - 2, mb0, ss0)
                work(k, set0, mb0)
                scat_fire(k, mb0, ss0)

            @pl.when(jnp.logical_not(even))
            def _():
                drain(k, set1, sb)

                @pl.when(k < CPW - 1)
                def _():
                    fire(k + 1, set0, sa)

                @pl.when(k >= 2)
                def _():
                    scat_drain(k - 2, mb1, ss1)
                work(k, set1, mb1)
                scat_fire(k, mb1, ss1)
            return cy

        lax.fori_loop(0, CPW, pipe, 0)
        scat_drain(CPW - 2, mb0, ss0)
        scat_drain(CPW - 1, mb1, ss1)
        return wcy

    lax.fori_loop(0, NWIN, window_body, 0)
    plsc.subcore_barrier()

    def ocopy(t, cy):
        r0 = s * RPT + t * RC
        pltpu.sync_copy(shared.at[pl.ds(r0, RC)], sbuf)

        def rrow(i, icy):
            for g in range(4):
                ve, vo = plsc.unpack(sbuf[i, pl.ds(g * 32, 32)],
                                     format=plsc.PackFormat.INTERLEAVED)
                obuf[i, pl.ds(g * 32, 16)] = jnp.maximum(ve, 0.0)
                obuf[i, pl.ds(g * 32 + 16, 16)] = jnp.maximum(vo, 0.0)
            return icy
        lax.fori_loop(0, RC, rrow, 0)
        pltpu.sync_copy(obuf, out.at[pl.ds(r0, RC), pl.ds(coff, 128)])
        return cy
    lax.fori_loop(0, NRC, ocopy, 0)


def _agg_pass(ls, lr, sub, rel, obj, alpha):
    mesh = plsc.VectorSubcoreMesh(core_axis_name="c", subcore_axis_name="s")
    f = pl.kernel(
        _agg_body,
        out_type=jax.ShapeDtypeStruct((N_NODES, D), jnp.float32),
        mesh=mesh,
        scratch_types=[
            pltpu.VMEM_SHARED((N_NODES, 128), jnp.bfloat16),
            pltpu.VMEM((WIN,), jnp.int32),
            pltpu.VMEM((WIN,), jnp.int32),
            pltpu.VMEM((WIN,), jnp.int32),
            pltpu.VMEM((WIN + 16,), jnp.float32),
            pltpu.VMEM((B2, 128), jnp.bfloat16),
            pltpu.VMEM((B2, 128), jnp.bfloat16),
            pltpu.VMEM((B2, 128), jnp.bfloat16),
            pltpu.VMEM((B2, 128), jnp.bfloat16),
            pltpu.VMEM((B2, 128), jnp.bfloat16),
            pltpu.VMEM((B2, 128), jnp.bfloat16),
            pltpu.VMEM((RC, 128), jnp.float32),
            pltpu.VMEM((RC, 128), jnp.bfloat16),
            pltpu.SemaphoreType.DMA,
            pltpu.SemaphoreType.DMA,
            pltpu.SemaphoreType.DMA,
            pltpu.SemaphoreType.DMA,
        ],
        compiler_params=pltpu.CompilerParams(needs_layout_passes=False, use_tc_tiling_on_sc=False),
    )
    return f(ls, lr, sub, rel, obj, alpha)


def kernel(q_rel, layer_input, edges, nodes, n_ent, rela_embed,
           Ws_attn, Wr_attn, Wqr_attn_W, Wqr_attn_b,
           w_alpha_W, w_alpha_b, W_h):
    sub = edges[:, 4]
    rel = edges[:, 2]
    obj = edges[:, 5]
    ridx = edges[:, 0]

    xl = jnp.pad(layer_input, ((0, MP - layer_input.shape[0]), (0, 0)))
    xr = jnp.pad(rela_embed, ((0, MP - rela_embed.shape[0]), (0, 0)))

    bf = jnp.bfloat16
    ps = _mm(xl, Ws_attn, bf)
    pr = _mm(xr, Wr_attn, bf)
    pq = _mm_bias(xr, Wqr_attn_W, Wqr_attn_b, bf)
    whp = W_h[:, _PERM]
    ls = _mm_colsplit(xl, whp, bf)
    lr = _mm_colsplit(xr, whp, bf)

    wcat = jnp.concatenate(
        [w_alpha_W[_EO, 0], w_alpha_b, jnp.zeros((15,), jnp.float32)])

    alpha = _alpha_pass(ps, pr, pq, sub, rel, ridx, q_rel, wcat)
    return _agg_pass(ls, lr, sub, rel, obj, alpha)


# speedup vs baseline: 4.4707x; 1.0241x over previous
import functools

import jax
import jax.numpy as jnp
import numpy as np
from jax import lax
from jax.experimental import pallas as pl
from jax.experimental.pallas import tpu as pltpu
from jax.experimental.pallas import tpu_sc as plsc

N_NODES = 10000
N_QUERIES = 10000
N_EDGES = 160000
D = 256
MP = 10240
NC, NS = 2, 16
NW = NC * NS

EPW = N_EDGES // NW
B1 = 40
NCH1 = EPW // B1

EPT = N_EDGES // NS
B2 = 40
WIN = 2000
NWIN = EPT // WIN
CPW = WIN // B2
RPT = N_NODES // NS
RC = 25
NRC = RPT // RC

_PERM = np.empty((D,), np.int32)
for _g in range(D // 32):
    for _t in range(16):
        _PERM[32 * _g + 2 * _t] = 32 * _g + _t
        _PERM[32 * _g + 2 * _t + 1] = 32 * _g + 16 + _t
_EO = np.empty((D,), np.int32)
for _g in range(D // 32):
    _EO[32 * _g:32 * _g + 16] = np.arange(32 * _g, 32 * _g + 32, 2)
    _EO[32 * _g + 16:32 * _g + 32] = np.arange(32 * _g + 1, 32 * _g + 32, 2)


def _mm_kernel(x_ref, w_ref, o_ref):
    o_ref[...] = jnp.dot(x_ref[...], w_ref[...],
                         preferred_element_type=jnp.float32
                         ).astype(o_ref.dtype)


def _mm_bias_kernel(x_ref, w_ref, b_ref, o_ref):
    o_ref[...] = (jnp.dot(x_ref[...], w_ref[...],
                          preferred_element_type=jnp.float32)
                  + b_ref[0:1, :]).astype(o_ref.dtype)


def _mm(x, w, out_dtype=jnp.float32, m=None):
    k = x.shape[1]
    m = x.shape[0] if m is None else m
    n = w.shape[1]
    bm = 1024
    return pl.pallas_call(
        _mm_kernel,
        grid=(m // bm,),
        in_specs=[pl.BlockSpec((bm, k), lambda i: (i, 0)),
                  pl.BlockSpec((k, n), lambda i: (0, 0))],
        out_specs=pl.BlockSpec((bm, n), lambda i: (i, 0)),
        out_shape=jax.ShapeDtypeStruct((m, n), out_dtype),
    )(x, w)


def _mm_bias(x, w, b, out_dtype=jnp.float32, m=None):
    k = x.shape[1]
    m = x.shape[0] if m is None else m
    n = w.shape[1]
    bm = 1024
    b8 = jnp.zeros((8, n), jnp.float32).at[0].set(b)
    return pl.pallas_call(
        _mm_bias_kernel,
        grid=(m // bm,),
        in_specs=[pl.BlockSpec((bm, k), lambda i: (i, 0)),
                  pl.BlockSpec((k, n), lambda i: (0, 0)),
                  pl.BlockSpec((8, n), lambda i: (0, 0))],
        out_specs=pl.BlockSpec((bm, n), lambda i: (i, 0)),
        out_shape=jax.ShapeDtypeStruct((m, n), out_dtype),
    )(x, w, b8)


def _mm_colsplit(x, w, out_dtype=jnp.float32, m=None):
    k = x.shape[1]
    m = x.shape[0] if m is None else m
    bm = 1024
    nb = m // bm
    return pl.pallas_call(
        _mm_kernel,
        grid=(nb, 2),
        in_specs=[pl.BlockSpec((bm, k), lambda i, j: (i, 0)),
                  pl.BlockSpec((k, 128), lambda i, j: (0, j))],
        out_specs=pl.BlockSpec((bm, 128), lambda i, j: (j * nb + i, 0)),
        out_shape=jax.ShapeDtypeStruct((2 * m, 128), out_dtype),
    )(x, w)


def _alpha_body(ps, pr, pq, sub, rel, ridx, qrel, wcat, alpha_out,
                isub, irel, irx, iqr, qtab, ra0, rb0, rc0, ra1, rb1, rc1,
                wv, dots, sa, sb):
    c = lax.axis_index("c")
    s = lax.axis_index("s")
    wid = s * NC + c
    base0 = wid * EPW
    pltpu.sync_copy(wcat, wv)
    b2 = wv[pl.ds(256, 16)][0]
    wregs = [wv[pl.ds(j * 16, 16)] for j in range(16)]
    lane = lax.iota(jnp.int32, 16)
    m15 = lane == 15

    irx[pl.ds(EPW - 8, 16)] = jnp.zeros((16,), jnp.int32)
    pltpu.sync_copy(sub.at[pl.ds(base0, EPW)], isub)
    pltpu.sync_copy(rel.at[pl.ds(base0, EPW)], irel)
    pltpu.sync_copy(ridx.at[pl.ds(base0, EPW)], irx.at[pl.ds(0, EPW)])
    pltpu.sync_copy(qrel, qtab)

    def qcomp(g, cy):
        sl = pl.ds(g * 16, 16)
        iqr[sl] = plsc.load_gather(qtab, [irx[sl]])
        return cy
    lax.fori_loop(0, (EPW + 8) // 16, qcomp, 0)

    def fire(kk, bufs, sem):
        st = kk * B1
        ra, rb, rc_ = bufs
        pltpu.async_copy(ps.at[isub.at[pl.ds(st, B1)]], ra, sem)
        pltpu.async_copy(pr.at[irel.at[pl.ds(st, B1)]], rb, sem)
        pltpu.async_copy(pq.at[iqr.at[pl.ds(st, B1)]], rc_, sem)

    def drain(kk, bufs, sem):
        st = kk * B1
        ra, rb, rc_ = bufs
        pltpu.make_async_copy(ps.at[isub.at[pl.ds(st, B1)]], ra, sem).wait()
        pltpu.make_async_copy(pr.at[irel.at[pl.ds(st, B1)]], rb, sem).wait()
        pltpu.make_async_copy(pq.at[iqr.at[pl.ds(st, B1)]], rc_, sem).wait()

    def compute(kk, bufs):
        ra, rb, rc_ = bufs

        def edge_body(e, cy):
            acc = jnp.zeros((16,), jnp.float32)
            for g in range(16 // 2):
                sl = pl.ds(g * 32, 32)
                s32 = ra[e, sl] + rb[e, sl] + rc_[e, sl]
                ve, vo = plsc.unpack(s32, format=plsc.PackFormat.INTERLEAVED)
                acc = acc + jnp.maximum(ve, 0.0) * wregs[2 * g]
                acc = acc + jnp.maximum(vo, 0.0) * wregs[2 * g + 1]
            tot = plsc.cumsum(acc)
            plsc.store_scatter(dots, [jnp.full((16,), kk * B1 + e, jnp.int32)],
                               tot, mask=m15)
            return cy

        lax.fori_loop(0, B1, edge_body, 0)

    set0 = (ra0, rb0, rc0)
    set1 = (ra1, rb1, rc1)
    fire(0, set0, sa)

    def pipe(g, cy):
        k0 = g * 2
        drain(k0, set0, sa)
        fire(k0 + 1, set1, sb)
        compute(k0, set0)
        drain(k0 + 1, set1, sb)

        @pl.when(g < NCH1 // 2 - 1)
        def _():
            fire(k0 + 2, set0, sa)
        compute(k0 + 1, set1)
        return cy

    lax.fori_loop(0, NCH1 // 2, pipe, 0)
    fire(NCH1 - 1, set0, sa)
    drain(NCH1 - 1, set0, sa)
    compute(NCH1 - 1, set0)

    def sig(g, cy):
        sl = pl.ds(g * 16, 16)
        v = dots[sl]
        dots[sl] = 1.0 / (1.0 + jnp.exp(-(v + b2)))
        return cy
    lax.fori_loop(0, (EPW + 8) // 16, sig, 0)
    pltpu.sync_copy(dots.at[pl.ds(0, EPW)], alpha_out.at[pl.ds(base0, EPW)])


def _alpha_pass(ps, pr, pq, sub, rel, ridx, qrel, wcat):
    mesh = plsc.VectorSubcoreMesh(core_axis_name="c", subcore_axis_name="s")
    f = pl.kernel(
        _alpha_body,
        out_type=jax.ShapeDtypeStruct((N_EDGES,), jnp.float32),
        mesh=mesh,
        scratch_types=[
            pltpu.VMEM((EPW,), jnp.int32),
            pltpu.VMEM((EPW,), jnp.int32),
            pltpu.VMEM((EPW + 8,), jnp.int32),
            pltpu.VMEM((EPW + 8,), jnp.int32),
            pltpu.VMEM((N_QUERIES,), jnp.int32),
            pltpu.VMEM((B1, D), jnp.bfloat16),
            pltpu.VMEM((B1, D), jnp.bfloat16),
            pltpu.VMEM((B1, D), jnp.bfloat16),
            pltpu.VMEM((B1, D), jnp.bfloat16),
            pltpu.VMEM((B1, D), jnp.bfloat16),
            pltpu.VMEM((B1, D), jnp.bfloat16),
            pltpu.VMEM((272,), jnp.float32),
            pltpu.VMEM((EPW + 8,), jnp.float32),
            pltpu.SemaphoreType.DMA,
            pltpu.SemaphoreType.DMA,
        ],
        compiler_params=pltpu.CompilerParams(needs_layout_passes=False, use_tc_tiling_on_sc=False),
    )
    return f(ps, pr, pq, sub, rel, ridx, qrel, wcat)


def _agg_body(ls, lr, sub, rel, obj, alpha, out,
              shared, isub, irel, iobj, abuf, ga, gb, gc, gd, mb0, mb1,
              obuf, sbuf, sa, sb, ss0, ss1):
    c = lax.axis_index("c")
    s = lax.axis_index("s")
    coff = c * 128
    roff = c * MP

    def zrow(i, cy):
        for j in range(4):
            sbuf[i, pl.ds(j * 32, 32)] = jnp.zeros((32,), jnp.bfloat16)
        return cy
    lax.fori_loop(0, RC, zrow, 0)

    def zcopy(t, cy):
        pltpu.sync_copy(sbuf, shared.at[pl.ds(s * RPT + t * RC, RC)])
        return cy
    lax.fori_loop(0, NRC, zcopy, 0)
    plsc.subcore_barrier()

    base0 = s * EPT

    def load_window(w):
        wb = base0 + w * WIN
        pltpu.sync_copy(sub.at[pl.ds(wb, WIN)], isub)
        pltpu.sync_copy(rel.at[pl.ds(wb, WIN)], irel)
        pltpu.sync_copy(obj.at[pl.ds(wb, WIN)], iobj)
        pltpu.sync_copy(alpha.at[pl.ds(wb, WIN)], abuf.at[pl.ds(0, WIN)])

        def offs(i, cy):
            sl = pl.ds(i * 16, 16)
            isub[sl] = isub[sl] + roff
            irel[sl] = irel[sl] + roff
            return cy
        lax.fori_loop(0, WIN // 16, offs, 0)

    def fire(kk, bufs, sem):
        st = kk * B2
        ga_, gb_ = bufs
        pltpu.async_copy(ls.at[isub.at[pl.ds(st, B2)]], ga_, sem)
        pltpu.async_copy(lr.at[irel.at[pl.ds(st, B2)]], gb_, sem)

    def drain(kk, bufs, sem):
        st = kk * B2
        ga_, gb_ = bufs
        pltpu.make_async_copy(ls.at[isub.at[pl.ds(st, B2)]], ga_, sem).wait()
        pltpu.make_async_copy(lr.at[irel.at[pl.ds(st, B2)]], gb_, sem).wait()

    def work(kk, bufs, mb):
        st = kk * B2
        ga_, gb_ = bufs

        def do_edge(e, a):
            for g in range(4):
                sl = pl.ds(g * 32, 32)
                s32 = ga_[e, sl] + gb_[e, sl]
                ve, vo = plsc.unpack(s32, format=plsc.PackFormat.INTERLEAVED)
                mb[e, sl] = plsc.pack(ve * a, vo * a,
                                      format=plsc.PackFormat.INTERLEAVED)

        def group_body(g, cy):
            av = abuf[pl.ds(st + g * 16, 16)]
            for l in range(16):
                do_edge(g * 16 + l, av[l])
            return cy

        lax.fori_loop(0, B2 // 16, group_body, 0)
        av = abuf[pl.ds(st + 32, 16)]
        for l in range(8):
            do_edge(32 + l, av[l])

    def scat_fire(kk, mb, sem):
        pltpu.async_copy(mb, shared.at[iobj.at[pl.ds(kk * B2, B2)]], sem,
                         add=True)

    def scat_drain(kk, mb, sem):
        pltpu.make_async_copy(mb, shared.at[iobj.at[pl.ds(kk * B2, B2)]],
                              sem).wait()

    set0 = (ga, gb)
    set1 = (gc, gd)

    def window_body(w, wcy):
        load_window(w)
        fire(0, set0, sa)

        def pipe(k, cy):
            even = lax.rem(k, 2) == 0

            @pl.when(even)
            def _():
                drain(k, set0, sa)

                @pl.when(k < CPW - 1)
                def _():
                    fire(k + 1, set1, sb)

                @pl.when(k >= 2)
                def _():
                    scat_drain(k - 2, mb0, ss0)
                work(k, set0, mb0)
                scat_fire(k, mb0, ss0)

            @pl.when(jnp.logical_not(even))
            def _():
                drain(k, set1, sb)

                @pl.when(k < CPW - 1)
                def _():
                    fire(k + 1, set0, sa)

                @pl.when(k >= 2)
                def _():
                    scat_drain(k - 2, mb1, ss1)
                work(k, set1, mb1)
                scat_fire(k, mb1, ss1)
            return cy

        lax.fori_loop(0, CPW, pipe, 0)
        scat_drain(CPW - 2, mb0, ss0)
        scat_drain(CPW - 1, mb1, ss1)
        return wcy

    lax.fori_loop(0, NWIN, window_body, 0)
    plsc.subcore_barrier()

    def ocopy(t, cy):
        r0 = s * RPT + t * RC
        pltpu.sync_copy(shared.at[pl.ds(r0, RC)], sbuf)

        def rrow(i, icy):
            for g in range(4):
                ve, vo = plsc.unpack(sbuf[i, pl.ds(g * 32, 32)],
                                     format=plsc.PackFormat.INTERLEAVED)
                obuf[i, pl.ds(g * 32, 16)] = jnp.maximum(ve, 0.0)
                obuf[i, pl.ds(g * 32 + 16, 16)] = jnp.maximum(vo, 0.0)
            return icy
        lax.fori_loop(0, RC, rrow, 0)
        pltpu.sync_copy(obuf, out.at[pl.ds(r0, RC), pl.ds(coff, 128)])
        return cy
    lax.fori_loop(0, NRC, ocopy, 0)


def _agg_pass(ls, lr, sub, rel, obj, alpha):
    mesh = plsc.VectorSubcoreMesh(core_axis_name="c", subcore_axis_name="s")
    f = pl.kernel(
        _agg_body,
        out_type=jax.ShapeDtypeStruct((N_NODES, D), jnp.float32),
        mesh=mesh,
        scratch_types=[
            pltpu.VMEM_SHARED((N_NODES, 128), jnp.bfloat16),
            pltpu.VMEM((WIN,), jnp.int32),
            pltpu.VMEM((WIN,), jnp.int32),
            pltpu.VMEM((WIN,), jnp.int32),
            pltpu.VMEM((WIN + 16,), jnp.float32),
            pltpu.VMEM((B2, 128), jnp.bfloat16),
            pltpu.VMEM((B2, 128), jnp.bfloat16),
            pltpu.VMEM((B2, 128), jnp.bfloat16),
            pltpu.VMEM((B2, 128), jnp.bfloat16),
            pltpu.VMEM((B2, 128), jnp.bfloat16),
            pltpu.VMEM((B2, 128), jnp.bfloat16),
            pltpu.VMEM((RC, 128), jnp.float32),
            pltpu.VMEM((RC, 128), jnp.bfloat16),
            pltpu.SemaphoreType.DMA,
            pltpu.SemaphoreType.DMA,
            pltpu.SemaphoreType.DMA,
            pltpu.SemaphoreType.DMA,
        ],
        compiler_params=pltpu.CompilerParams(needs_layout_passes=False, use_tc_tiling_on_sc=False),
    )
    return f(ls, lr, sub, rel, obj, alpha)


def kernel(q_rel, layer_input, edges, nodes, n_ent, rela_embed,
           Ws_attn, Wr_attn, Wqr_attn_W, Wqr_attn_b,
           w_alpha_W, w_alpha_b, W_h):
    sub = edges[:, 4]
    rel = edges[:, 2]
    obj = edges[:, 5]
    ridx = edges[:, 0]

    bf = jnp.bfloat16
    ps = _mm(layer_input, Ws_attn, bf, m=MP)
    pr = _mm(rela_embed, Wr_attn, bf, m=MP)
    pq = _mm_bias(rela_embed, Wqr_attn_W, Wqr_attn_b, bf, m=MP)
    whp = W_h[:, _PERM]
    ls = _mm_colsplit(layer_input, whp, bf, m=MP)
    lr = _mm_colsplit(rela_embed, whp, bf, m=MP)

    wcat = jnp.concatenate(
        [w_alpha_W[_EO, 0], w_alpha_b, jnp.zeros((15,), jnp.float32)])

    alpha = _alpha_pass(ps, pr, pq, sub, rel, ridx, q_rel, wcat)
    return _agg_pass(ls, lr, sub, rel, obj, alpha)
